# baseline (speedup 1.0000x reference)
import functools

import jax
import jax.numpy as jnp
from jax import lax
from jax.experimental import pallas as pl
from jax.experimental.pallas import tpu as pltpu
from jax.experimental.pallas import tpu_sc as plsc

N = 10000
NP = 10240
E = 320000
CHUNK = 80
NC = 2
NS = 16
NW = NC * NS
CPT = 128
NCT = NW * CPT
EPAD = NCT * CHUNK
H = 8
F = 128
D = 64
DW = 128
D2 = 48
NPAIR = 4
NCLASS = 40
VW = 16
SW = 8
ALPHA = 0.2
RPT = NP // NS
ZB = RPT // CHUNK


def _bcast(v16, lane):
    idx = jnp.full((16, 1), lane, jnp.int32)
    dnums = lax.GatherDimensionNumbers(
        offset_dims=(), collapsed_slice_dims=(0,), start_index_map=(0,))
    return lax.gather(v16, idx, dnums, (1,),
                      mode=lax.GatherScatterMode.PROMISE_IN_BOUNDS)


def _mesh():
    return plsc.VectorSubcoreMesh(core_axis_name="c", subcore_axis_name="s",
                                  num_cores=NC, num_subcores=NS)


def _edge_vals(srs_v, drs_v, c1, c2, j, lane_iota):
    rr = j * 16 + lane_iota
    s1g = plsc.load_gather(srs_v, [rr, jnp.full((16,), c1, jnp.int32)])
    s2g = plsc.load_gather(drs_v, [rr, jnp.full((16,), c2, jnp.int32)])
    t = s1g + s2g
    return jnp.exp(-jnp.where(t > 0, t, ALPHA * t))


def _zero_locals(rows_v, vst_v, width):
    zero16 = jnp.zeros((16,), jnp.float32)

    def zrow(i, _):
        for j in range(width // 16):
            rows_v[i, pl.ds(j * 16, 16)] = zero16
        vst_v[i, :] = zero16
        return 0
    lax.fori_loop(0, CHUNK, zrow, 0)


def _mkident(idb_v, base, lane_iota):
    def body(j, _):
        idb_v[0, pl.ds(j * 16, 16)] = base + j * 16 + lane_iota
        return 0
    lax.fori_loop(0, CHUNK // 16, body, 0)


def _make_sc_l1():
    CT = NCT // NS

    @functools.partial(
        pl.kernel,
        out_type=(
            jax.ShapeDtypeStruct((H, NP, D), jnp.float32),
            jax.ShapeDtypeStruct((H, NP, VW), jnp.float32),
        ),
        mesh=_mesh(),
        compiler_params=pltpu.CompilerParams(needs_layout_passes=False),
        scratch_types=[
            pltpu.VMEM((8, CHUNK), jnp.int32),
            pltpu.VMEM((8, CHUNK), jnp.int32),
            pltpu.VMEM((CHUNK,), jnp.int32),
            pltpu.VMEM((ZB, CHUNK), jnp.int32),
            pltpu.VMEM((CHUNK, DW), jnp.float32),
            pltpu.VMEM((CHUNK, D), jnp.float32),
            pltpu.VMEM((CHUNK, SW), jnp.float32),
            pltpu.VMEM((CHUNK, SW), jnp.float32),
            pltpu.VMEM((CHUNK, VW), jnp.float32),
            pltpu.VMEM((CHUNK,), jnp.float32),
            pltpu.SemaphoreType.DMA,
            pltpu.SemaphoreType.DMA,
            pltpu.SemaphoreType.DMA,
            pltpu.VMEM_SHARED((NP, SW), jnp.float32),
            pltpu.VMEM_SHARED((NP, D), jnp.float32),
            pltpu.VMEM_SHARED((NP, VW), jnp.float32),
        ],
    )
    def l1(src_hbm, dst_hbm, tab_hbm, srow_hbm, hacc_out, vacc_out,
           src_v, dst_v, idx_v, idb_v, rows_v, half_v, srs_v, drs_v, vst_v,
           val_v, sem_r, sem_s, sem_d, stab_s, acc_s, vacc_s):
        cid = lax.axis_index("c")
        sid = lax.axis_index("s")
        lane_iota = lax.iota(jnp.int32, 16)
        for z in range(ZB):
            _mkident(idb_v.at[pl.ds(z, 1)], sid * RPT + z * CHUNK, lane_iota)

        def pair_body(pr, _):
            pair = cid * 2 + pr
            for z in range(ZB):
                base = sid * RPT + z * CHUNK
                pltpu.sync_copy(
                    srow_hbm.at[pair, pl.ds(base, CHUNK), :], srs_v)
                pltpu.sync_copy(srs_v, stab_s.at[idb_v.at[z]])
            for q in range(2):
                _zero_locals(half_v, vst_v, D)
                for z in range(ZB):
                    pltpu.sync_copy(half_v, acc_s.at[idb_v.at[z]])
                    pltpu.sync_copy(vst_v, vacc_s.at[idb_v.at[z]])
                plsc.subcore_barrier()

                def group_body(cg, _):
                    grow = sid * CT + cg * 8
                    pltpu.sync_copy(src_hbm.at[pl.ds(grow, 8), :], src_v)
                    pltpu.sync_copy(dst_hbm.at[pl.ds(grow, 8), :], dst_v)

                    def chunk_body(k, _):
                        def idx_body(j, _):
                            dv = dst_v[k, pl.ds(j * 16, 16)]
                            idx_v[pl.ds(j * 16, 16)] = dv + pair * NP
                            return 0
                        lax.fori_loop(0, CHUNK // 16, idx_body, 0)

                        cp_r = pltpu.async_copy(tab_hbm.at[idx_v], rows_v,
                                                sem_r)
                        cp_s = pltpu.async_copy(stab_s.at[src_v.at[k]], srs_v,
                                                sem_s)
                        cp_d = pltpu.async_copy(stab_s.at[dst_v.at[k]], drs_v,
                                                sem_d)
                        cp_s.wait()
                        cp_d.wait()

                        def val_body(j, _):
                            ve = _edge_vals(srs_v, drs_v, 2 * q, 2 * q + 1,
                                            j, lane_iota)
                            val_v[pl.ds(j * 16, 16)] = ve
                            plsc.store_scatter(
                                vst_v,
                                [j * 16 + lane_iota,
                                 jnp.zeros((16,), jnp.int32)], ve)
                            return 0
                        lax.fori_loop(0, CHUNK // 16, val_body, 0)
                        cp_r.wait()

                        def scale_body(g, _):
                            v16 = val_v[pl.ds(g * 16, 16)]
                            for j in range(16):
                                e = g * 16 + j
                                vb = _bcast(v16, j)
                                for kk in range(D // 16):
                                    half_v[e, pl.ds(kk * 16, 16)] = (
                                        rows_v[e, pl.ds(q * D + kk * 16, 16)]
                                        * vb)
                            return 0
                        lax.fori_loop(0, CHUNK // 16, scale_body, 0)

                        pltpu.sync_copy(half_v, acc_s.at[src_v.at[k]],
                                        add=True)
                        pltpu.sync_copy(vst_v, vacc_s.at[src_v.at[k]],
                                        add=True)
                        return 0
                    lax.fori_loop(0, 8, chunk_body, 0)
                    return 0
                lax.fori_loop(0, CT // 8, group_body, 0)
                plsc.subcore_barrier()

                head = pair * 2 + q
                for z in range(ZB):
                    base = sid * RPT + z * CHUNK
                    pltpu.sync_copy(acc_s.at[idb_v.at[z]], half_v)
                    pltpu.sync_copy(
                        half_v, hacc_out.at[head, pl.ds(base, CHUNK), :])
                    pltpu.sync_copy(vacc_s.at[idb_v.at[z]], vst_v)
                    pltpu.sync_copy(
                        vst_v, vacc_out.at[head, pl.ds(base, CHUNK), :])
            return 0
        lax.fori_loop(0, 2, pair_body, 0)

    return l1


def _make_sc_l2():
    @functools.partial(
        pl.kernel,
        out_type=(
            jax.ShapeDtypeStruct((NC, NP, D2), jnp.float32),
            jax.ShapeDtypeStruct((NC, NP, VW), jnp.float32),
        ),
        mesh=_mesh(),
        compiler_params=pltpu.CompilerParams(needs_layout_passes=False),
        scratch_types=[
            pltpu.VMEM((8, CHUNK), jnp.int32),
            pltpu.VMEM((8, CHUNK), jnp.int32),
            pltpu.VMEM((ZB, CHUNK), jnp.int32),
            pltpu.VMEM((CHUNK, DW), jnp.float32),
            pltpu.VMEM((CHUNK, D2), jnp.float32),
            pltpu.VMEM((CHUNK, SW), jnp.float32),
            pltpu.VMEM((CHUNK, SW), jnp.float32),
            pltpu.VMEM((CHUNK, VW), jnp.float32),
            pltpu.VMEM((CHUNK,), jnp.float32),
            pltpu.SemaphoreType.DMA,
            pltpu.SemaphoreType.DMA,
            pltpu.SemaphoreType.DMA,
            pltpu.VMEM_SHARED((NP, SW), jnp.float32),
            pltpu.VMEM_SHARED((NP, D2), jnp.float32),
            pltpu.VMEM_SHARED((NP, VW), jnp.float32),
        ],
    )
    def l2(src_hbm, dst_hbm, tab_hbm, srow_hbm, gacc_out, vacc_out,
           src_v, dst_v, idb_v, rows_v, out_v, srs_v, drs_v, vst_v, val_v,
           sem_r, sem_s, sem_d, stab_s, acc_s, vacc_s):
        cid = lax.axis_index("c")
        sid = lax.axis_index("s")
        gtid = cid * NS + sid
        lane_iota = lax.iota(jnp.int32, 16)
        for z in range(ZB):
            _mkident(idb_v.at[pl.ds(z, 1)], sid * RPT + z * CHUNK, lane_iota)

        _zero_locals(out_v, vst_v, D2)
        for z in range(ZB):
            base = sid * RPT + z * CHUNK
            pltpu.sync_copy(srow_hbm.at[0, pl.ds(base, CHUNK), :], srs_v)
            pltpu.sync_copy(srs_v, stab_s.at[idb_v.at[z]])
            pltpu.sync_copy(out_v, acc_s.at[idb_v.at[z]])
            pltpu.sync_copy(vst_v, vacc_s.at[idb_v.at[z]])
        plsc.subcore_barrier()

        def group_body(cg, _):
            grow = gtid * CPT + cg * 8
            pltpu.sync_copy(src_hbm.at[pl.ds(grow, 8), :], src_v)
            pltpu.sync_copy(dst_hbm.at[pl.ds(grow, 8), :], dst_v)

            def chunk_body(k, _):
                cp_r = pltpu.async_copy(tab_hbm.at[dst_v.at[k]], rows_v,
                                        sem_r)
                cp_s = pltpu.async_copy(stab_s.at[src_v.at[k]], srs_v, sem_s)
                cp_d = pltpu.async_copy(stab_s.at[dst_v.at[k]], drs_v, sem_d)
                cp_s.wait()
                cp_d.wait()

                def val_body(j, _):
                    ve = _edge_vals(srs_v, drs_v, 0, 1, j, lane_iota)
                    val_v[pl.ds(j * 16, 16)] = ve
                    plsc.store_scatter(
                        vst_v,
                        [j * 16 + lane_iota, jnp.zeros((16,), jnp.int32)], ve)
                    return 0
                lax.fori_loop(0, CHUNK // 16, val_body, 0)
                cp_r.wait()

                def scale_body(g, _):
                    v16 = val_v[pl.ds(g * 16, 16)]
                    for j in range(16):
                        e = g * 16 + j
                        vb = _bcast(v16, j)
                        for kk in range(D2 // 16):
                            out_v[e, pl.ds(kk * 16, 16)] = (
                                rows_v[e, pl.ds(kk * 16, 16)] * vb)
                    return 0
                lax.fori_loop(0, CHUNK // 16, scale_body, 0)

                pltpu.sync_copy(out_v, acc_s.at[src_v.at[k]], add=True)
                pltpu.sync_copy(vst_v, vacc_s.at[src_v.at[k]], add=True)
                return 0
            lax.fori_loop(0, 8, chunk_body, 0)
            return 0
        lax.fori_loop(0, CPT // 8, group_body, 0)
        plsc.subcore_barrier()

        for z in range(ZB):
            base = sid * RPT + z * CHUNK
            pltpu.sync_copy(acc_s.at[idb_v.at[z]], out_v)
            pltpu.sync_copy(out_v, gacc_out.at[cid, pl.ds(base, CHUNK), :])
            pltpu.sync_copy(vacc_s.at[idb_v.at[z]], vst_v)
            pltpu.sync_copy(vst_v, vacc_out.at[cid, pl.ds(base, CHUNK), :])

    return l2


_make_sc_l1 = functools.cache(_make_sc_l1)
_make_sc_l2 = functools.cache(_make_sc_l2)


BN1 = 2000


def _tc1_body(x_ref, w_ref, a_ref, h_ref, s_ref):
    xv = x_ref[...]
    ha = jnp.dot(xv, w_ref[0], preferred_element_type=jnp.float32)
    hb = jnp.dot(xv, w_ref[1], preferred_element_type=jnp.float32)
    h_ref[0] = jnp.concatenate([ha, hb], axis=1)
    s1a = jnp.sum(ha * a_ref[0, 0, :D][None, :], axis=1)
    s2a = jnp.sum(ha * a_ref[0, 0, D:][None, :], axis=1)
    s1b = jnp.sum(hb * a_ref[1, 0, :D][None, :], axis=1)
    s2b = jnp.sum(hb * a_ref[1, 0, D:][None, :], axis=1)
    s_ref[0] = jnp.stack([s1a, s2a, s1b, s2b], axis=1)


def _tc_stage1(x, Ws, As):
    return pl.pallas_call(
        _tc1_body,
        grid=(NPAIR, N // BN1),
        in_specs=[
            pl.BlockSpec((BN1, F), lambda p, i: (i, 0)),
            pl.BlockSpec((2, F, D), lambda p, i: (p, 0, 0)),
            pl.BlockSpec((2, 1, 2 * D), lambda p, i: (p, 0, 0)),
        ],
        out_specs=[
            pl.BlockSpec((1, BN1, DW), lambda p, i: (p, i, 0)),
            pl.BlockSpec((1, BN1, 4), lambda p, i: (p, i, 0)),
        ],
        out_shape=[
            jax.ShapeDtypeStruct((NPAIR, N, DW), jnp.float32),
            jax.ShapeDtypeStruct((NPAIR, N, 4), jnp.float32),
        ],
    )(x, Ws, As)


BN3 = 2560


def _tc3_body(hp_ref, vp_ref, w_ref, ao_ref, g_ref, so_ref):
    acc = jnp.zeros((BN3, DW), jnp.float32)
    for h in range(H):
        rs = vp_ref[h, :, 0:1]
        hpn = hp_ref[h] / (rs + 1e-16)
        xh = jnp.where(hpn > 0, hpn, jnp.exp(jnp.minimum(hpn, 0.0)) - 1.0)
        acc = acc + jnp.dot(xh, w_ref[h], preferred_element_type=jnp.float32)
    g_ref[...] = acc
    s1 = jnp.sum(acc * ao_ref[0, :DW][None, :], axis=1)
    s2 = jnp.sum(acc * ao_ref[0, DW:][None, :], axis=1)
    so_ref[...] = jnp.stack([s1, s2], axis=1)


def _tc_stage3(hacc, vacc, w_pad, ao_pad):
    return pl.pallas_call(
        _tc3_body,
        grid=(NP // BN3,),
        in_specs=[
            pl.BlockSpec((H, BN3, D), lambda i: (0, i, 0)),
            pl.BlockSpec((H, BN3, VW), lambda i: (0, i, 0)),
            pl.BlockSpec((H, D, DW), lambda i: (0, 0, 0)),
            pl.BlockSpec((1, 2 * DW), lambda i: (0, 0)),
        ],
        out_specs=[
            pl.BlockSpec((BN3, DW), lambda i: (i, 0)),
            pl.BlockSpec((BN3, 2), lambda i: (i, 0)),
        ],
        out_shape=[
            jax.ShapeDtypeStruct((NP, DW), jnp.float32),
            jax.ShapeDtypeStruct((NP, 2), jnp.float32),
        ],
    )(hacc, vacc, w_pad, ao_pad)


BN5 = 2560


def _tc5_body(gp_ref, vp_ref, o_ref):
    g = gp_ref[0, :, :NCLASS] + gp_ref[1, :, :NCLASS]
    rs = vp_ref[0, :, 0:1] + vp_ref[1, :, 0:1]
    h2 = g / (rs + 1e-16)
    o = jnp.where(h2 > 0, h2, jnp.exp(jnp.minimum(h2, 0.0)) - 1.0)
    m = jnp.max(o, axis=1, keepdims=True)
    z = o - m
    lse = jnp.log(jnp.sum(jnp.exp(z), axis=1, keepdims=True))
    o_ref[...] = z - lse


def _tc_stage5(gacc, vacc2):
    return pl.pallas_call(
        _tc5_body,
        grid=(NP // BN5,),
        in_specs=[
            pl.BlockSpec((NC, BN5, D2), lambda i: (0, i, 0)),
            pl.BlockSpec((NC, BN5, VW), lambda i: (0, i, 0)),
        ],
        out_specs=pl.BlockSpec((BN5, NCLASS), lambda i: (i, 0)),
        out_shape=jax.ShapeDtypeStruct((NP, NCLASS), jnp.float32),
    )(gacc, vacc2)


def kernel(x, adj, Ws, As, W_out, a_out):
    src = adj[0]
    dst = adj[1]
    padv = jnp.full((EPAD - E,), N, jnp.int32)
    srcp = jnp.concatenate([src, padv]).reshape(NCT, CHUNK)
    dstp = jnp.concatenate([dst, padv]).reshape(NCT, CHUNK)

    h2, s12 = _tc_stage1(x, Ws, As)
    h_tab = jnp.pad(h2, ((0, 0), (0, NP - N), (0, 0))).reshape(NPAIR * NP, DW)
    srows1 = jnp.pad(s12, ((0, 0), (0, NP - N), (0, SW - 4)))

    hacc, vacc = _make_sc_l1()(srcp, dstp, h_tab, srows1)

    w_pad = jnp.pad(W_out.reshape(H, D, NCLASS),
                    ((0, 0), (0, 0), (0, DW - NCLASS)))
    ao_pad = jnp.zeros((1, 2 * DW), jnp.float32)
    ao_pad = ao_pad.at[0, :NCLASS].set(a_out[0, :NCLASS])
    ao_pad = ao_pad.at[0, DW:DW + NCLASS].set(a_out[0, NCLASS:])

    g_tab, so = _tc_stage3(hacc, vacc, w_pad, ao_pad)
    srows2 = jnp.pad(so[None], ((0, 0), (0, 0), (0, SW - 2)))

    gacc, vacc2 = _make_sc_l2()(srcp, dstp, g_tab, srows2)

    outp = _tc_stage5(gacc, vacc2)
    return outp[:N]


# speedup vs baseline: 1.1162x; 1.1162x over previous
import functools

import jax
import jax.numpy as jnp
from jax import lax
from jax.experimental import pallas as pl
from jax.experimental.pallas import tpu as pltpu
from jax.experimental.pallas import tpu_sc as plsc

N = 10000
NP = 10240
E = 320000
CHUNK = 80
NC = 2
NS = 16
NW = NC * NS
CPT = 126
NCT = NW * CPT
EPAD = NCT * CHUNK
H = 8
F = 128
D = 64
DW = 128
D2 = 48
NPAIR = 4
NCLASS = 40
VW = 16
SW = 8
ALPHA = 0.2
RPT = NP // NS
ZB = RPT // CHUNK


def _bcast(v16, lane):
    idx = jnp.full((16, 1), lane, jnp.int32)
    dnums = lax.GatherDimensionNumbers(
        offset_dims=(), collapsed_slice_dims=(0,), start_index_map=(0,))
    return lax.gather(v16, idx, dnums, (1,),
                      mode=lax.GatherScatterMode.PROMISE_IN_BOUNDS)


def _mesh():
    return plsc.VectorSubcoreMesh(core_axis_name="c", subcore_axis_name="s",
                                  num_cores=NC, num_subcores=NS)


def _edge_vals(srs_v, drs_v, c1, c2, j, lane_iota):
    rr = j * 16 + lane_iota
    s1g = plsc.load_gather(srs_v, [rr, jnp.full((16,), c1, jnp.int32)])
    s2g = plsc.load_gather(drs_v, [rr, jnp.full((16,), c2, jnp.int32)])
    t = s1g + s2g
    return jnp.exp(-jnp.where(t > 0, t, ALPHA * t))


def _zero_locals(rows_v, vst_v, width):
    zero16 = jnp.zeros((16,), jnp.float32)

    def zrow(i, _):
        for j in range(width // 16):
            rows_v[i, pl.ds(j * 16, 16)] = zero16
        vst_v[i, :] = zero16
        return 0
    lax.fori_loop(0, CHUNK, zrow, 0)


def _mkident(idb_v, base, lane_iota):
    def body(j, _):
        idb_v[0, pl.ds(j * 16, 16)] = base + j * 16 + lane_iota
        return 0
    lax.fori_loop(0, CHUNK // 16, body, 0)


def _make_sc_l1():
    CT = NCT // NS

    @functools.partial(
        pl.kernel,
        out_type=(
            jax.ShapeDtypeStruct((H, NP, D), jnp.float32),
            jax.ShapeDtypeStruct((H, NP, VW), jnp.float32),
        ),
        mesh=_mesh(),
        compiler_params=pltpu.CompilerParams(needs_layout_passes=False),
        scratch_types=[
            pltpu.VMEM((1, CHUNK), jnp.int32),
            pltpu.VMEM((1, CHUNK), jnp.int32),
            pltpu.VMEM((CHUNK,), jnp.int32),
            pltpu.VMEM((ZB, CHUNK), jnp.int32),
            pltpu.VMEM((CHUNK, DW), jnp.float32),
            pltpu.VMEM((CHUNK, D), jnp.float32),
            pltpu.VMEM((CHUNK, SW), jnp.float32),
            pltpu.VMEM((CHUNK, SW), jnp.float32),
            pltpu.VMEM((CHUNK, VW), jnp.float32),
            pltpu.VMEM((CHUNK,), jnp.float32),
            pltpu.SemaphoreType.DMA,
            pltpu.SemaphoreType.DMA,
            pltpu.SemaphoreType.DMA,
            pltpu.VMEM_SHARED((NP, SW), jnp.float32),
            pltpu.VMEM_SHARED((NP, D), jnp.float32),
            pltpu.VMEM_SHARED((NP, VW), jnp.float32),
        ],
    )
    def l1(src_hbm, dst_hbm, tab_hbm, srow_hbm, hacc_out, vacc_out,
           src_v, dst_v, idx_v, idb_v, rows_v, half_v, srs_v, drs_v, vst_v,
           val_v, sem_r, sem_s, sem_d, stab_s, acc_s, vacc_s):
        cid = lax.axis_index("c")
        sid = lax.axis_index("s")
        lane_iota = lax.iota(jnp.int32, 16)
        for z in range(ZB):
            _mkident(idb_v.at[pl.ds(z, 1)], sid * RPT + z * CHUNK, lane_iota)

        def pair_body(pr, _):
            pair = cid * 2 + pr
            for z in range(ZB):
                base = sid * RPT + z * CHUNK
                pltpu.sync_copy(
                    srow_hbm.at[pair, pl.ds(base, CHUNK), :], srs_v)
                pltpu.sync_copy(srs_v, stab_s.at[idb_v.at[z]])
            for q in range(2):
                _zero_locals(half_v, vst_v, D)
                for z in range(ZB):
                    pltpu.sync_copy(half_v, acc_s.at[idb_v.at[z]])
                    pltpu.sync_copy(vst_v, vacc_s.at[idb_v.at[z]])
                plsc.subcore_barrier()

                def chunk_body(ci, _):
                    row = sid * CT + ci
                    pltpu.sync_copy(src_hbm.at[pl.ds(row, 1), :], src_v)
                    pltpu.sync_copy(dst_hbm.at[pl.ds(row, 1), :], dst_v)

                    def idx_body(j, _):
                        dv = dst_v[0, pl.ds(j * 16, 16)]
                        idx_v[pl.ds(j * 16, 16)] = dv + pair * NP
                        return 0
                    lax.fori_loop(0, CHUNK // 16, idx_body, 0)

                    cp_r = pltpu.async_copy(tab_hbm.at[idx_v], rows_v,
                                            sem_r)
                    cp_s = pltpu.async_copy(stab_s.at[src_v.at[0]], srs_v,
                                            sem_s)
                    cp_d = pltpu.async_copy(stab_s.at[dst_v.at[0]], drs_v,
                                            sem_d)
                    cp_s.wait()
                    cp_d.wait()

                    def val_body(j, _):
                        ve = _edge_vals(srs_v, drs_v, 2 * q, 2 * q + 1,
                                        j, lane_iota)
                        val_v[pl.ds(j * 16, 16)] = ve
                        plsc.store_scatter(
                            vst_v,
                            [j * 16 + lane_iota,
                             jnp.zeros((16,), jnp.int32)], ve)
                        return 0
                    lax.fori_loop(0, CHUNK // 16, val_body, 0)
                    cp_r.wait()

                    def scale_body(g, _):
                        v16 = val_v[pl.ds(g * 16, 16)]
                        for j in range(16):
                            e = g * 16 + j
                            vb = _bcast(v16, j)
                            for k in range(D // 16):
                                half_v[e, pl.ds(k * 16, 16)] = (
                                    rows_v[e, pl.ds(q * D + k * 16, 16)] * vb)
                        return 0
                    lax.fori_loop(0, CHUNK // 16, scale_body, 0)

                    pltpu.sync_copy(half_v, acc_s.at[src_v.at[0]], add=True)
                    pltpu.sync_copy(vst_v, vacc_s.at[src_v.at[0]], add=True)
                    return 0
                lax.fori_loop(0, CT, chunk_body, 0)
                plsc.subcore_barrier()

                head = pair * 2 + q
                for z in range(ZB):
                    base = sid * RPT + z * CHUNK
                    pltpu.sync_copy(acc_s.at[idb_v.at[z]], half_v)
                    pltpu.sync_copy(
                        half_v, hacc_out.at[head, pl.ds(base, CHUNK), :])
                    pltpu.sync_copy(vacc_s.at[idb_v.at[z]], vst_v)
                    pltpu.sync_copy(
                        vst_v, vacc_out.at[head, pl.ds(base, CHUNK), :])
            return 0
        lax.fori_loop(0, 2, pair_body, 0)

    return l1


def _make_sc_l2():
    @functools.partial(
        pl.kernel,
        out_type=(
            jax.ShapeDtypeStruct((NC, NP, D2), jnp.float32),
            jax.ShapeDtypeStruct((NC, NP, VW), jnp.float32),
        ),
        mesh=_mesh(),
        compiler_params=pltpu.CompilerParams(needs_layout_passes=False),
        scratch_types=[
            pltpu.VMEM((1, CHUNK), jnp.int32),
            pltpu.VMEM((1, CHUNK), jnp.int32),
            pltpu.VMEM((ZB, CHUNK), jnp.int32),
            pltpu.VMEM((CHUNK, DW), jnp.float32),
            pltpu.VMEM((CHUNK, D2), jnp.float32),
            pltpu.VMEM((CHUNK, SW), jnp.float32),
            pltpu.VMEM((CHUNK, SW), jnp.float32),
            pltpu.VMEM((CHUNK, VW), jnp.float32),
            pltpu.VMEM((CHUNK,), jnp.float32),
            pltpu.SemaphoreType.DMA,
            pltpu.SemaphoreType.DMA,
            pltpu.SemaphoreType.DMA,
            pltpu.VMEM_SHARED((NP, SW), jnp.float32),
            pltpu.VMEM_SHARED((NP, D2), jnp.float32),
            pltpu.VMEM_SHARED((NP, VW), jnp.float32),
        ],
    )
    def l2(src_hbm, dst_hbm, tab_hbm, srow_hbm, gacc_out, vacc_out,
           src_v, dst_v, idb_v, rows_v, out_v, srs_v, drs_v, vst_v, val_v,
           sem_r, sem_s, sem_d, stab_s, acc_s, vacc_s):
        cid = lax.axis_index("c")
        sid = lax.axis_index("s")
        gtid = cid * NS + sid
        lane_iota = lax.iota(jnp.int32, 16)
        for z in range(ZB):
            _mkident(idb_v.at[pl.ds(z, 1)], sid * RPT + z * CHUNK, lane_iota)

        _zero_locals(out_v, vst_v, D2)
        for z in range(ZB):
            base = sid * RPT + z * CHUNK
            pltpu.sync_copy(srow_hbm.at[0, pl.ds(base, CHUNK), :], srs_v)
            pltpu.sync_copy(srs_v, stab_s.at[idb_v.at[z]])
            pltpu.sync_copy(out_v, acc_s.at[idb_v.at[z]])
            pltpu.sync_copy(vst_v, vacc_s.at[idb_v.at[z]])
        plsc.subcore_barrier()

        def chunk_body(ci, _):
            row = gtid * CPT + ci
            pltpu.sync_copy(src_hbm.at[pl.ds(row, 1), :], src_v)
            pltpu.sync_copy(dst_hbm.at[pl.ds(row, 1), :], dst_v)

            cp_r = pltpu.async_copy(tab_hbm.at[dst_v.at[0]], rows_v, sem_r)
            cp_s = pltpu.async_copy(stab_s.at[src_v.at[0]], srs_v, sem_s)
            cp_d = pltpu.async_copy(stab_s.at[dst_v.at[0]], drs_v, sem_d)
            cp_s.wait()
            cp_d.wait()

            def val_body(j, _):
                ve = _edge_vals(srs_v, drs_v, 0, 1, j, lane_iota)
                val_v[pl.ds(j * 16, 16)] = ve
                plsc.store_scatter(
                    vst_v,
                    [j * 16 + lane_iota, jnp.zeros((16,), jnp.int32)], ve)
                return 0
            lax.fori_loop(0, CHUNK // 16, val_body, 0)
            cp_r.wait()

            def scale_body(g, _):
                v16 = val_v[pl.ds(g * 16, 16)]
                for j in range(16):
                    e = g * 16 + j
                    vb = _bcast(v16, j)
                    for k in range(D2 // 16):
                        out_v[e, pl.ds(k * 16, 16)] = (
                            rows_v[e, pl.ds(k * 16, 16)] * vb)
                return 0
            lax.fori_loop(0, CHUNK // 16, scale_body, 0)

            pltpu.sync_copy(out_v, acc_s.at[src_v.at[0]], add=True)
            pltpu.sync_copy(vst_v, vacc_s.at[src_v.at[0]], add=True)
            return 0
        lax.fori_loop(0, CPT, chunk_body, 0)
        plsc.subcore_barrier()

        for z in range(ZB):
            base = sid * RPT + z * CHUNK
            pltpu.sync_copy(acc_s.at[idb_v.at[z]], out_v)
            pltpu.sync_copy(out_v, gacc_out.at[cid, pl.ds(base, CHUNK), :])
            pltpu.sync_copy(vacc_s.at[idb_v.at[z]], vst_v)
            pltpu.sync_copy(vst_v, vacc_out.at[cid, pl.ds(base, CHUNK), :])

    return l2


_make_sc_l1 = functools.cache(_make_sc_l1)
_make_sc_l2 = functools.cache(_make_sc_l2)


BN1 = 2000


def _tc1_body(x_ref, w_ref, a_ref, h_ref, s_ref):
    xv = x_ref[...]
    ha = jnp.dot(xv, w_ref[0], preferred_element_type=jnp.float32)
    hb = jnp.dot(xv, w_ref[1], preferred_element_type=jnp.float32)
    h_ref[0] = jnp.concatenate([ha, hb], axis=1)
    s1a = jnp.sum(ha * a_ref[0, 0, :D][None, :], axis=1)
    s2a = jnp.sum(ha * a_ref[0, 0, D:][None, :], axis=1)
    s1b = jnp.sum(hb * a_ref[1, 0, :D][None, :], axis=1)
    s2b = jnp.sum(hb * a_ref[1, 0, D:][None, :], axis=1)
    s_ref[0] = jnp.stack([s1a, s2a, s1b, s2b], axis=1)


def _tc_stage1(x, Ws, As):
    return pl.pallas_call(
        _tc1_body,
        grid=(NPAIR, N // BN1),
        in_specs=[
            pl.BlockSpec((BN1, F), lambda p, i: (i, 0)),
            pl.BlockSpec((2, F, D), lambda p, i: (p, 0, 0)),
            pl.BlockSpec((2, 1, 2 * D), lambda p, i: (p, 0, 0)),
        ],
        out_specs=[
            pl.BlockSpec((1, BN1, DW), lambda p, i: (p, i, 0)),
            pl.BlockSpec((1, BN1, 4), lambda p, i: (p, i, 0)),
        ],
        out_shape=[
            jax.ShapeDtypeStruct((NPAIR, N, DW), jnp.float32),
            jax.ShapeDtypeStruct((NPAIR, N, 4), jnp.float32),
        ],
    )(x, Ws, As)


BN3 = 2560


def _tc3_body(hp_ref, vp_ref, w_ref, ao_ref, g_ref, so_ref):
    acc = jnp.zeros((BN3, DW), jnp.float32)
    for h in range(H):
        rs = vp_ref[h, :, 0:1]
        hpn = hp_ref[h] / (rs + 1e-16)
        xh = jnp.where(hpn > 0, hpn, jnp.exp(jnp.minimum(hpn, 0.0)) - 1.0)
        acc = acc + jnp.dot(xh, w_ref[h], preferred_element_type=jnp.float32)
    g_ref[...] = acc
    s1 = jnp.sum(acc * ao_ref[0, :DW][None, :], axis=1)
    s2 = jnp.sum(acc * ao_ref[0, DW:][None, :], axis=1)
    so_ref[...] = jnp.stack([s1, s2], axis=1)


def _tc_stage3(hacc, vacc, w_pad, ao_pad):
    return pl.pallas_call(
        _tc3_body,
        grid=(NP // BN3,),
        in_specs=[
            pl.BlockSpec((H, BN3, D), lambda i: (0, i, 0)),
            pl.BlockSpec((H, BN3, VW), lambda i: (0, i, 0)),
            pl.BlockSpec((H, D, DW), lambda i: (0, 0, 0)),
            pl.BlockSpec((1, 2 * DW), lambda i: (0, 0)),
        ],
        out_specs=[
            pl.BlockSpec((BN3, DW), lambda i: (i, 0)),
            pl.BlockSpec((BN3, 2), lambda i: (i, 0)),
        ],
        out_shape=[
            jax.ShapeDtypeStruct((NP, DW), jnp.float32),
            jax.ShapeDtypeStruct((NP, 2), jnp.float32),
        ],
    )(hacc, vacc, w_pad, ao_pad)


BN5 = 2560


def _tc5_body(gp_ref, vp_ref, o_ref):
    g = gp_ref[0, :, :NCLASS] + gp_ref[1, :, :NCLASS]
    rs = vp_ref[0, :, 0:1] + vp_ref[1, :, 0:1]
    h2 = g / (rs + 1e-16)
    o = jnp.where(h2 > 0, h2, jnp.exp(jnp.minimum(h2, 0.0)) - 1.0)
    m = jnp.max(o, axis=1, keepdims=True)
    z = o - m
    lse = jnp.log(jnp.sum(jnp.exp(z), axis=1, keepdims=True))
    o_ref[...] = z - lse


def _tc_stage5(gacc, vacc2):
    return pl.pallas_call(
        _tc5_body,
        grid=(NP // BN5,),
        in_specs=[
            pl.BlockSpec((NC, BN5, D2), lambda i: (0, i, 0)),
            pl.BlockSpec((NC, BN5, VW), lambda i: (0, i, 0)),
        ],
        out_specs=pl.BlockSpec((BN5, NCLASS), lambda i: (i, 0)),
        out_shape=jax.ShapeDtypeStruct((NP, NCLASS), jnp.float32),
    )(gacc, vacc2)


def kernel(x, adj, Ws, As, W_out, a_out):
    src = adj[0]
    dst = adj[1]
    padv = jnp.full((EPAD - E,), N, jnp.int32)
    srcp = jnp.concatenate([src, padv]).reshape(NCT, CHUNK)
    dstp = jnp.concatenate([dst, padv]).reshape(NCT, CHUNK)

    h2, s12 = _tc_stage1(x, Ws, As)
    h_tab = jnp.pad(h2, ((0, 0), (0, NP - N), (0, 0))).reshape(NPAIR * NP, DW)
    srows1 = jnp.pad(s12, ((0, 0), (0, NP - N), (0, SW - 4)))

    hacc, vacc = _make_sc_l1()(srcp, dstp, h_tab, srows1)

    w_pad = jnp.pad(W_out.reshape(H, D, NCLASS),
                    ((0, 0), (0, 0), (0, DW - NCLASS)))
    ao_pad = jnp.zeros((1, 2 * DW), jnp.float32)
    ao_pad = ao_pad.at[0, :NCLASS].set(a_out[0, :NCLASS])
    ao_pad = ao_pad.at[0, DW:DW + NCLASS].set(a_out[0, NCLASS:])

    g_tab, so = _tc_stage3(hacc, vacc, w_pad, ao_pad)
    srows2 = jnp.pad(so[None], ((0, 0), (0, 0), (0, SW - 2)))

    gacc, vacc2 = _make_sc_l2()(srcp, dstp, g_tab, srows2)

    outp = _tc_stage5(gacc, vacc2)
    return outp[:N]
